# baseline (speedup 1.0000x reference)
import jax
import jax.numpy as jnp
from jax.experimental import pallas as pl
from jax.experimental.pallas import tpu as pltpu

B, T, F, H, C = 4, 256, 64, 128, 2
K = 4


def _rows(scalars, width=H):
    return jnp.concatenate(
        [jnp.full((1, width), s, jnp.float32) for s in scalars], axis=0)


def _body(m_s, x_s, d_s, len_s, wd_s, bd_s,
          vm_t, ng_t, W0, W1, bl, WoT, bo_s, out_ref, h_ref, outs_ref):
    h_ref[...] = jnp.zeros((B, F, H), dtype=jnp.float32)
    n_steps = jnp.maximum(jnp.maximum(len_s[0], len_s[1]),
                          jnp.maximum(len_s[2], len_s[3]))
    n_blocks = (n_steps + (K - 1)) // K

    def matvec_one(j, b, fi):
        dv = wd_s[fi] * d_s[b, j] + bd_s[fi]
        decay = jnp.exp(-jnp.maximum(jnp.full((1, H), dv), 0.0))
        h_bf = (decay * h_ref[b, pl.ds(fi, 1), :]).astype(jnp.bfloat16)
        out = jax.lax.dot_general(
            h_bf, W1[fi], (((1,), (0,)), ((), ())),
            preferred_element_type=jnp.float32)
        return out + x_s[b, j] * W0[pl.ds(fi, 1), :] + bl[pl.ds(fi, 1), :]

    def block(i, carry):
        c_all, s_all, cnt_all = carry
        j0 = i * K
        fis = [[m_s[b, j0 + k] for b in range(B)] for k in range(K)]

        for k in range(K):
            for b in range(B):
                outs_ref[pl.ds(k * B + b, 1), :] = \
                    matvec_one(j0 + k, b, fis[k][b])

        for k in range(K):
            j = j0 + k
            for b in range(B):
                for kp in range(k):
                    dup = fis[k][b] == fis[kp][b]
                    if kp == 0:
                        dup_b = dup
                    else:
                        dup_b = dup_b | dup
                if k > 0:
                    @pl.when(dup_b)
                    def _():
                        outs_ref[pl.ds(k * B + b, 1), :] = \
                            matvec_one(j, b, fis[k][b])

            vmask = vm_t[j]
            ngmask = ng_t[j]

            c_all = ngmask * (s_all / jnp.maximum(cnt_all, 1.0)) \
                + (1.0 - ngmask) * c_all
            s_all = (1.0 - ngmask) * s_all
            cnt_all = (1.0 - ngmask) * cnt_all

            outs = outs_ref[pl.ds(k * B, B), :]
            sg = jax.nn.sigmoid(outs[:, 0:3 * H])
            gi = sg[:, 0:H]
            gf = sg[:, H:2 * H]
            go = sg[:, 2 * H:3 * H]
            gc = jnp.tanh(outs[:, 3 * H:4 * H])
            cell = gf * c_all + gi * gc
            hnew = go * jnp.tanh(cell)
            for b in range(B):
                h_old = h_ref[b, pl.ds(fis[k][b], 1), :]
                h_ref[b, pl.ds(fis[k][b], 1), :] = \
                    vmask[b:b + 1] * hnew[b:b + 1] \
                    + (1.0 - vmask[b:b + 1]) * h_old
            s_all = s_all + vmask * cell
            cnt_all = cnt_all + vmask
        return c_all, s_all, cnt_all

    init = (jnp.zeros((B, H), jnp.float32),
            jnp.zeros((B, H), jnp.float32),
            jnp.zeros((B, H), jnp.float32))
    c_all, s_all, cnt_all = jax.lax.fori_loop(0, n_blocks, block, init)
    c_all = s_all / cnt_all

    for b in range(B):
        feat = jnp.concatenate([c_all[b:b + 1], h_ref[b]], axis=0)
        l0 = jnp.sum(feat * WoT[0])
        l1 = jnp.sum(feat * WoT[1])
        d = (l1 - l0) + (bo_s[1] - bo_s[0])
        p1 = jax.nn.sigmoid(jnp.full((1, H), d))
        out_ref[b:b + 1, 0:1] = (1.0 - p1)[:, 0:1]
        out_ref[b:b + 1, 1:2] = p1[:, 0:1]


def kernel(X, lengths, W_l, b_l, w_d, b_d, W_o, b_o):
    t = X[:, 0, :]
    m = X[:, 1, :].astype(jnp.int32)
    x = X[:, 2, :]
    delt = X[:, 3, :]
    lengths = lengths.astype(jnp.int32)
    W0 = W_l[:, 0, :]
    W1 = W_l[:, 1:, :].astype(jnp.bfloat16)
    WoT = W_o.reshape(F + 1, H, C).transpose(2, 0, 1)

    j_idx = jnp.arange(T)[None, :]
    valid = j_idx < lengths[:, None]
    ng = valid & (j_idx > 0) & (t != jnp.roll(t, 1, axis=1))
    vm_t = jnp.broadcast_to(
        valid.T.astype(jnp.float32)[:, :, None], (T, B, H))
    ng_t = jnp.broadcast_to(
        ng.T.astype(jnp.float32)[:, :, None], (T, B, H))

    smem = pl.BlockSpec(memory_space=pltpu.SMEM)
    vmem = pl.BlockSpec(memory_space=pltpu.VMEM)
    out = pl.pallas_call(
        _body,
        out_shape=jax.ShapeDtypeStruct((B, C), jnp.float32),
        in_specs=[smem, smem, smem, smem, smem, smem,
                  vmem, vmem, vmem, vmem, vmem, vmem, smem],
        out_specs=pl.BlockSpec(memory_space=pltpu.VMEM),
        scratch_shapes=[pltpu.VMEM((B, F, H), jnp.float32),
                        pltpu.VMEM((K * B, 4 * H), jnp.float32)],
    )(m, x, delt, lengths, w_d, b_d,
      vm_t, ng_t, W0, W1, b_l, WoT, b_o)
    return out


# speedup vs baseline: 1.0218x; 1.0218x over previous
import jax
import jax.numpy as jnp
from jax.experimental import pallas as pl
from jax.experimental.pallas import tpu as pltpu

B, T, F, H, C = 4, 256, 64, 128, 2
K = 4


def _rows(scalars, width=H):
    return jnp.concatenate(
        [jnp.full((1, width), s, jnp.float32) for s in scalars], axis=0)


def _body(m_s, x_s, d_s, len_s, wd_s, bd_s,
          vm_t, ng_t, W0, W1, bl, WoT, bo_s, out_ref, h_ref, outs_ref):
    h_ref[...] = jnp.zeros((B, F, H), dtype=jnp.float32)
    n_steps = jnp.maximum(jnp.maximum(len_s[0], len_s[1]),
                          jnp.maximum(len_s[2], len_s[3]))
    n_blocks = (n_steps + (K - 1)) // K

    def matvec_one(j, b, fi):
        dv = wd_s[fi] * d_s[b, j] + bd_s[fi]
        decay = jnp.exp(-jnp.maximum(jnp.full((1, H), dv), 0.0))
        h_bf = (decay * h_ref[b, pl.ds(fi, 1), :]).astype(jnp.bfloat16)
        out = jax.lax.dot_general(
            h_bf, W1[fi], (((1,), (0,)), ((), ())),
            preferred_element_type=jnp.float32)
        return out + x_s[b, j] * W0[pl.ds(fi, 1), :] + bl[pl.ds(fi, 1), :]

    def block(i, carry):
        c_all, s_all, cnt_all = carry
        j0 = i * K
        fis = [[m_s[b, j0 + k] for b in range(B)] for k in range(K)]

        for k in range(K):
            for b in range(B):
                outs_ref[pl.ds(k * B + b, 1), :] = \
                    matvec_one(j0 + k, b, fis[k][b])

        for k in range(K):
            j = j0 + k
            if k > 0:
                dup = None
                for b in range(B):
                    for kp in range(k):
                        d_b = fis[k][b] == fis[kp][b]
                        dup = d_b if dup is None else (dup | d_b)

                @pl.when(dup)
                def _():
                    for b in range(B):
                        outs_ref[pl.ds(k * B + b, 1), :] = \
                            matvec_one(j, b, fis[k][b])

            vmask = vm_t[j]
            ngmask = ng_t[j]

            c_all = ngmask * (s_all / jnp.maximum(cnt_all, 1.0)) \
                + (1.0 - ngmask) * c_all
            s_all = (1.0 - ngmask) * s_all
            cnt_all = (1.0 - ngmask) * cnt_all

            outs = outs_ref[pl.ds(k * B, B), :]
            sg = jax.nn.sigmoid(outs[:, 0:3 * H])
            gi = sg[:, 0:H]
            gf = sg[:, H:2 * H]
            go = sg[:, 2 * H:3 * H]
            gc = jnp.tanh(outs[:, 3 * H:4 * H])
            cell = gf * c_all + gi * gc
            hnew = go * jnp.tanh(cell)
            for b in range(B):
                h_old = h_ref[b, pl.ds(fis[k][b], 1), :]
                h_ref[b, pl.ds(fis[k][b], 1), :] = \
                    vmask[b:b + 1] * hnew[b:b + 1] \
                    + (1.0 - vmask[b:b + 1]) * h_old
            s_all = s_all + vmask * cell
            cnt_all = cnt_all + vmask
        return c_all, s_all, cnt_all

    init = (jnp.zeros((B, H), jnp.float32),
            jnp.zeros((B, H), jnp.float32),
            jnp.zeros((B, H), jnp.float32))
    c_all, s_all, cnt_all = jax.lax.fori_loop(0, n_blocks, block, init)
    c_all = s_all / cnt_all

    for b in range(B):
        feat = jnp.concatenate([c_all[b:b + 1], h_ref[b]], axis=0)
        l0 = jnp.sum(feat * WoT[0])
        l1 = jnp.sum(feat * WoT[1])
        d = (l1 - l0) + (bo_s[1] - bo_s[0])
        p1 = jax.nn.sigmoid(jnp.full((1, H), d))
        out_ref[b:b + 1, 0:1] = (1.0 - p1)[:, 0:1]
        out_ref[b:b + 1, 1:2] = p1[:, 0:1]


def kernel(X, lengths, W_l, b_l, w_d, b_d, W_o, b_o):
    t = X[:, 0, :]
    m = X[:, 1, :].astype(jnp.int32)
    x = X[:, 2, :]
    delt = X[:, 3, :]
    lengths = lengths.astype(jnp.int32)
    W0 = W_l[:, 0, :]
    W1 = W_l[:, 1:, :].astype(jnp.bfloat16)
    WoT = W_o.reshape(F + 1, H, C).transpose(2, 0, 1)

    j_idx = jnp.arange(T)[None, :]
    valid = j_idx < lengths[:, None]
    ng = valid & (j_idx > 0) & (t != jnp.roll(t, 1, axis=1))
    vm_t = jnp.broadcast_to(
        valid.T.astype(jnp.float32)[:, :, None], (T, B, H))
    ng_t = jnp.broadcast_to(
        ng.T.astype(jnp.float32)[:, :, None], (T, B, H))

    smem = pl.BlockSpec(memory_space=pltpu.SMEM)
    vmem = pl.BlockSpec(memory_space=pltpu.VMEM)
    out = pl.pallas_call(
        _body,
        out_shape=jax.ShapeDtypeStruct((B, C), jnp.float32),
        in_specs=[smem, smem, smem, smem, smem, smem,
                  vmem, vmem, vmem, vmem, vmem, vmem, smem],
        out_specs=pl.BlockSpec(memory_space=pltpu.VMEM),
        scratch_shapes=[pltpu.VMEM((B, F, H), jnp.float32),
                        pltpu.VMEM((K * B, 4 * H), jnp.float32)],
    )(m, x, delt, lengths, w_d, b_d,
      vm_t, ng_t, W0, W1, b_l, WoT, b_o)
    return out
